# baseline (speedup 1.0000x reference)
import functools

import jax
import jax.numpy as jnp
from jax import lax
from jax.experimental import pallas as pl
from jax.experimental.pallas import tpu as pltpu
from jax.experimental.pallas import tpu_sc as plsc

_NUM_TABLES = 26
_VOCAB = 100000
_DIM = 16
_NUM_IDX = 16384
_NC = 2
_NS = 16
_BINS = 102400
_VBLK = 12800
_NBLK = _BINS // _VBLK
_HALF = _NUM_TABLES // 2


def _make_sc_hist(off):
    def _sc_hist_body(lsi_hbm, out_hbm, idx_v, counts_v, sem):
        t = lax.axis_index("s") * _NC + lax.axis_index("c")

        @pl.when(t < _HALF)
        def _():
            h = pltpu.async_copy(lsi_hbm.at[t + off], idx_v, sem)

            zeros = jnp.zeros((16,), jnp.float32)

            def _zero(j, _):
                for u in range(8):
                    counts_v[pl.ds((j * 8 + u) * 16, 16)] = zeros
                return 0

            lax.fori_loop(0, _BINS // 16 // 8, _zero, 0)
            h.wait()

            def _hist(j, _):
                vs = [idx_v[(j * 8 + u) >> 3, pl.ds(((j * 8 + u) & 7) * 16, 16)]
                      for u in range(8)]
                cls = [plsc.scan_count(v) for v in vs]
                for v, (cnt, last) in zip(vs, cls):
                    plsc.addupdate_scatter(
                        counts_v, [v], cnt.astype(jnp.float32), mask=last)
                return 0

            lax.fori_loop(0, _NUM_IDX // 16 // 8, _hist, 0)
            pltpu.sync_copy(counts_v, out_hbm.at[pl.ds(t * _BINS, _BINS)])

    return functools.partial(
        pl.kernel,
        mesh=plsc.VectorSubcoreMesh(core_axis_name="c", subcore_axis_name="s"),
        out_type=jax.ShapeDtypeStruct((_HALF * _BINS,), jnp.float32),
        compiler_params=pltpu.CompilerParams(
            use_tc_tiling_on_sc=False, needs_layout_passes=False),
        scratch_types=[
            pltpu.VMEM((128, 128), jnp.int32),
            pltpu.VMEM((_BINS,), jnp.float32),
            pltpu.SemaphoreType.DMA,
        ],
    )(_sc_hist_body)


_sc_hist_lo = _make_sc_hist(0)
_sc_hist_hi = _make_sc_hist(_HALF)


def _tc_body(cnt_ref, tt_ref, out_ref):
    tb = tt_ref[0]
    acc = jnp.zeros((_DIM,), jnp.float32)
    for r in range(_NBLK):
        lo = r * _VBLK
        hi = min((r + 1) * _VBLK, _VOCAB)
        cbr = cnt_ref[0, pl.ds(r, 1), pl.ds(0, hi - lo)]
        acc += jnp.sum(tb[:, lo:hi] * cbr, axis=1)
    out_ref[0, 0, :] = acc


def _make_tc(off):
    return pl.pallas_call(
        _tc_body,
        grid=(_HALF,),
        in_specs=[
            pl.BlockSpec((1, _NBLK, _VBLK), lambda k: (k, 0, 0)),
            pl.BlockSpec((1, _DIM, _VOCAB), lambda k: (k + off, 0, 0)),
        ],
        out_specs=pl.BlockSpec((1, 1, _DIM), lambda k: (k, 0, 0)),
        out_shape=jax.ShapeDtypeStruct((_HALF, 1, _DIM), jnp.float32),
        compiler_params=pltpu.CompilerParams(
            dimension_semantics=("arbitrary",)),
    )


_tc_lo = _make_tc(0)
_tc_hi = _make_tc(_HALF)


@jax.jit
def _run(lS_i, tables):
    tt = jnp.swapaxes(tables, 1, 2)
    lsi3 = lS_i.reshape(_NUM_TABLES, 128, 128)
    c_lo = _sc_hist_lo(lsi3)
    c_hi = _sc_hist_hi(lsi3)
    o_lo = _tc_lo(c_lo.reshape(_HALF, _NBLK, _VBLK), tt)
    o_hi = _tc_hi(c_hi.reshape(_HALF, _NBLK, _VBLK), tt)
    return jnp.concatenate([o_lo, o_hi], axis=0).reshape(_NUM_TABLES, _DIM)


def kernel(lS_o, lS_i, tables):
    out = _run(lS_i, tables)
    return tuple(out[k:k + 1] for k in range(_NUM_TABLES))


# speedup vs baseline: 1.1010x; 1.1010x over previous
import functools

import jax
import jax.numpy as jnp
from jax import lax
from jax.experimental import pallas as pl
from jax.experimental.pallas import tpu as pltpu
from jax.experimental.pallas import tpu_sc as plsc

_NUM_TABLES = 26
_VOCAB = 100000
_DIM = 16
_NUM_IDX = 16384
_NC = 2
_NS = 16
_BINS = 102400
_VBLK = 12800
_NBLK = _BINS // _VBLK


def _sc_hist_body(lsi_hbm, out_hbm, idx_v, counts_v, sem):
    k = lax.axis_index("s") * _NC + lax.axis_index("c")

    @pl.when(k < _NUM_TABLES)
    def _():
        h = pltpu.async_copy(lsi_hbm.at[k], idx_v, sem)

        zeros = jnp.zeros((16,), jnp.float32)

        def _zero(j, _):
            for u in range(8):
                counts_v[pl.ds((j * 8 + u) * 16, 16)] = zeros
            return 0

        lax.fori_loop(0, _BINS // 16 // 8, _zero, 0)
        h.wait()

        def _hist(j, _):
            vs = [idx_v[(j * 8 + u) >> 3, pl.ds(((j * 8 + u) & 7) * 16, 16)]
                  for u in range(8)]
            cls = [plsc.scan_count(v) for v in vs]
            for v, (cnt, last) in zip(vs, cls):
                plsc.addupdate_scatter(
                    counts_v, [v], cnt.astype(jnp.float32), mask=last)
            return 0

        lax.fori_loop(0, _NUM_IDX // 16 // 8, _hist, 0)
        pltpu.sync_copy(counts_v, out_hbm.at[pl.ds(k * _BINS, _BINS)])


_sc_hist = functools.partial(
    pl.kernel,
    mesh=plsc.VectorSubcoreMesh(core_axis_name="c", subcore_axis_name="s"),
    out_type=jax.ShapeDtypeStruct((_NUM_TABLES * _BINS,), jnp.float32),
    compiler_params=pltpu.CompilerParams(
        use_tc_tiling_on_sc=False, needs_layout_passes=False),
    scratch_types=[
        pltpu.VMEM((128, 128), jnp.int32),
        pltpu.VMEM((_BINS,), jnp.float32),
        pltpu.SemaphoreType.DMA,
    ],
)(_sc_hist_body)


def _tc_body(cnt_ref, tt_ref, out_ref):
    tb = tt_ref[0]
    acc = jnp.zeros((_DIM,), jnp.float32)
    for r in range(_NBLK):
        lo = r * _VBLK
        hi = min((r + 1) * _VBLK, _VOCAB)
        cbr = cnt_ref[0, pl.ds(r, 1), pl.ds(0, hi - lo)]
        acc += jnp.sum(tb[:, lo:hi] * cbr, axis=1)
    out_ref[0, 0, :] = acc


_tc_reduce = pl.pallas_call(
    _tc_body,
    grid=(_NUM_TABLES,),
    in_specs=[
        pl.BlockSpec((1, _NBLK, _VBLK), lambda k: (k, 0, 0)),
        pl.BlockSpec((1, _DIM, _VOCAB), lambda k: (k, 0, 0)),
    ],
    out_specs=pl.BlockSpec((1, 1, _DIM), lambda k: (k, 0, 0)),
    out_shape=jax.ShapeDtypeStruct((_NUM_TABLES, 1, _DIM), jnp.float32),
    compiler_params=pltpu.CompilerParams(
        dimension_semantics=("arbitrary",)),
)


@jax.jit
def _run(lS_i, tables):
    tt = jnp.swapaxes(tables, 1, 2)
    lsi3 = lS_i.reshape(_NUM_TABLES, 128, 128)
    counts = _sc_hist(lsi3)
    cnt3 = counts.reshape(_NUM_TABLES, _NBLK, _VBLK)
    out = _tc_reduce(cnt3, tt)
    return out.reshape(_NUM_TABLES, _DIM)


def kernel(lS_o, lS_i, tables):
    out = _run(lS_i, tables)
    return tuple(out[k:k + 1] for k in range(_NUM_TABLES))


# speedup vs baseline: 1.1071x; 1.0055x over previous
import functools

import jax
import jax.numpy as jnp
from jax import lax
from jax.experimental import pallas as pl
from jax.experimental.pallas import tpu as pltpu
from jax.experimental.pallas import tpu_sc as plsc

_NUM_TABLES = 26
_VOCAB = 100000
_DIM = 16
_NUM_IDX = 16384
_NC = 2
_NS = 16
_BINS = 102400
_VBLK = 12800
_NBLK = _BINS // _VBLK


def _sc_hist_body(lsi_hbm, out_hbm, idx_v, counts_v, sem):
    k = lax.axis_index("s") * _NC + lax.axis_index("c")

    @pl.when(k < _NUM_TABLES)
    def _():
        h = pltpu.async_copy(lsi_hbm.at[k], idx_v, sem)

        zeros = jnp.zeros((16,), jnp.int32)

        def _zero(j, _):
            for u in range(8):
                counts_v[pl.ds((j * 8 + u) * 16, 16)] = zeros
            return 0

        lax.fori_loop(0, _BINS // 16 // 8, _zero, 0)
        h.wait()

        def _hist(j, _):
            vs = [idx_v[(j * 8 + u) >> 3, pl.ds(((j * 8 + u) & 7) * 16, 16)]
                  for u in range(8)]
            cls = [plsc.scan_count(v) for v in vs]
            for v, (cnt, last) in zip(vs, cls):
                plsc.addupdate_scatter(counts_v, [v], cnt, mask=last)
            return 0

        lax.fori_loop(0, _NUM_IDX // 16 // 8, _hist, 0)
        pltpu.sync_copy(counts_v, out_hbm.at[pl.ds(k * _BINS, _BINS)])


_sc_hist = functools.partial(
    pl.kernel,
    mesh=plsc.VectorSubcoreMesh(core_axis_name="c", subcore_axis_name="s"),
    out_type=jax.ShapeDtypeStruct((_NUM_TABLES * _BINS,), jnp.int32),
    compiler_params=pltpu.CompilerParams(
        use_tc_tiling_on_sc=False, needs_layout_passes=False),
    scratch_types=[
        pltpu.VMEM((128, 128), jnp.int32),
        pltpu.VMEM((_BINS,), jnp.int32),
        pltpu.SemaphoreType.DMA,
    ],
)(_sc_hist_body)


def _tc_body(cnt_ref, tt_ref, out_ref):
    tb = tt_ref[0]
    acc = jnp.zeros((_DIM,), jnp.float32)
    for r in range(_NBLK):
        lo = r * _VBLK
        hi = min((r + 1) * _VBLK, _VOCAB)
        cbr = cnt_ref[0, pl.ds(r, 1), pl.ds(0, hi - lo)]
        acc += jnp.sum(tb[:, lo:hi] * cbr.astype(jnp.float32), axis=1)
    out_ref[0, 0, :] = acc


_tc_reduce = pl.pallas_call(
    _tc_body,
    grid=(_NUM_TABLES,),
    in_specs=[
        pl.BlockSpec((1, _NBLK, _VBLK), lambda k: (k, 0, 0)),
        pl.BlockSpec((1, _DIM, _VOCAB), lambda k: (k, 0, 0)),
    ],
    out_specs=pl.BlockSpec((1, 1, _DIM), lambda k: (k, 0, 0)),
    out_shape=jax.ShapeDtypeStruct((_NUM_TABLES, 1, _DIM), jnp.float32),
    compiler_params=pltpu.CompilerParams(
        dimension_semantics=("arbitrary",)),
)


@jax.jit
def _run(lS_i, tables):
    tt = jnp.swapaxes(tables, 1, 2)
    lsi3 = lS_i.reshape(_NUM_TABLES, 128, 128)
    counts = _sc_hist(lsi3)
    cnt3 = counts.reshape(_NUM_TABLES, _NBLK, _VBLK)
    out = _tc_reduce(cnt3, tt)
    return out.reshape(_NUM_TABLES, _DIM)


def kernel(lS_o, lS_i, tables):
    out = _run(lS_i, tables)
    return tuple(out[k:k + 1] for k in range(_NUM_TABLES))


# speedup vs baseline: 1.1071x; 1.0000x over previous
import functools

import jax
import jax.numpy as jnp
from jax import lax
from jax.experimental import pallas as pl
from jax.experimental.pallas import tpu as pltpu
from jax.experimental.pallas import tpu_sc as plsc

_NUM_TABLES = 26
_VOCAB = 100000
_DIM = 16
_NUM_IDX = 16384
_NC = 2
_NS = 16
_BINS = 102400
_VBLK = 12800
_NBLK = _BINS // _VBLK


def _sc_hist_body(lsi_hbm, out_hbm, idx_v, counts_v, sem):
    k = lax.axis_index("s") * _NC + lax.axis_index("c")

    @pl.when(k < _NUM_TABLES)
    def _():
        h = pltpu.async_copy(lsi_hbm.at[k], idx_v, sem)

        zeros = jnp.zeros((16,), jnp.int32)

        def _zero(j, _):
            for u in range(8):
                counts_v[pl.ds((j * 8 + u) * 16, 16)] = zeros
            return 0

        lax.fori_loop(0, _BINS // 16 // 8, _zero, 0)
        h.wait()

        def _hist(j, _):
            vs = [idx_v[(j * 8 + u) >> 3, pl.ds(((j * 8 + u) & 7) * 16, 16)]
                  for u in range(8)]
            cls = [plsc.scan_count(v) for v in vs]
            for v, (cnt, last) in zip(vs, cls):
                plsc.addupdate_scatter(counts_v, [v], cnt, mask=last)
            return 0

        lax.fori_loop(0, _NUM_IDX // 16 // 8, _hist, 0)
        pltpu.sync_copy(counts_v, out_hbm.at[pl.ds(k * _BINS, _BINS)])


_sc_hist = functools.partial(
    pl.kernel,
    mesh=plsc.VectorSubcoreMesh(core_axis_name="c", subcore_axis_name="s"),
    out_type=jax.ShapeDtypeStruct((_NUM_TABLES * _BINS,), jnp.int32),
    compiler_params=pltpu.CompilerParams(
        use_tc_tiling_on_sc=False, needs_layout_passes=False),
    scratch_types=[
        pltpu.VMEM((128, 128), jnp.int32),
        pltpu.VMEM((_BINS,), jnp.int32),
        pltpu.SemaphoreType.DMA,
    ],
)(_sc_hist_body)


def _one_table(cnt_ref, tt_ref):
    tb = tt_ref[0]
    acc = jnp.zeros((_DIM,), jnp.float32)
    for r in range(_NBLK):
        lo = r * _VBLK
        hi = min((r + 1) * _VBLK, _VOCAB)
        cbr = cnt_ref[0, pl.ds(r, 1), pl.ds(0, hi - lo)]
        acc += jnp.sum(tb[:, lo:hi] * cbr.astype(jnp.float32), axis=1)
    return acc


def _tc_body(cnt_a, cnt_b, tt_a, tt_b, out_a, out_b):
    out_a[0, 0, :] = _one_table(cnt_a, tt_a)
    out_b[0, 0, :] = _one_table(cnt_b, tt_b)


_HALF = _NUM_TABLES // 2

_tc_reduce = pl.pallas_call(
    _tc_body,
    grid=(_HALF,),
    in_specs=[
        pl.BlockSpec((1, _NBLK, _VBLK), lambda k: (k, 0, 0)),
        pl.BlockSpec((1, _NBLK, _VBLK), lambda k: (k + _HALF, 0, 0)),
        pl.BlockSpec((1, _DIM, _VOCAB), lambda k: (k, 0, 0)),
        pl.BlockSpec((1, _DIM, _VOCAB), lambda k: (k + _HALF, 0, 0)),
    ],
    out_specs=[
        pl.BlockSpec((1, 1, _DIM), lambda k: (k, 0, 0)),
        pl.BlockSpec((1, 1, _DIM), lambda k: (k, 0, 0)),
    ],
    out_shape=[
        jax.ShapeDtypeStruct((_HALF, 1, _DIM), jnp.float32),
        jax.ShapeDtypeStruct((_HALF, 1, _DIM), jnp.float32),
    ],
    compiler_params=pltpu.CompilerParams(
        dimension_semantics=("arbitrary",)),
)


@jax.jit
def _run(lS_i, tables):
    tt = jnp.swapaxes(tables, 1, 2)
    lsi3 = lS_i.reshape(_NUM_TABLES, 128, 128)
    counts = _sc_hist(lsi3)
    cnt3 = counts.reshape(_NUM_TABLES, _NBLK, _VBLK)
    o_lo, o_hi = _tc_reduce(cnt3, cnt3, tt, tt)
    return jnp.concatenate([o_lo, o_hi], axis=0).reshape(_NUM_TABLES, _DIM)


def kernel(lS_o, lS_i, tables):
    out = _run(lS_i, tables)
    return tuple(out[k:k + 1] for k in range(_NUM_TABLES))


# speedup vs baseline: 1.1097x; 1.0023x over previous
import functools

import jax
import jax.numpy as jnp
from jax import lax
from jax.experimental import pallas as pl
from jax.experimental.pallas import tpu as pltpu
from jax.experimental.pallas import tpu_sc as plsc

_NUM_TABLES = 26
_VOCAB = 100000
_DIM = 16
_NUM_IDX = 16384
_NC = 2
_NS = 16
_BINS = 102400
_VBLK = 12800
_NBLK = _BINS // _VBLK


def _sc_hist_body(lsi_hbm, out_hbm, idx_v, counts_v, sem):
    k = lax.axis_index("s") * _NC + lax.axis_index("c")

    @pl.when(k < _NUM_TABLES)
    def _():
        h = pltpu.async_copy(lsi_hbm.at[k], idx_v, sem)

        zeros = jnp.zeros((16,), jnp.int32)

        def _zero(j, _):
            for u in range(8):
                counts_v[pl.ds((j * 8 + u) * 16, 16)] = zeros
            return 0

        lax.fori_loop(0, _BINS // 16 // 8, _zero, 0)
        h.wait()

        def _hist(j, _):
            vs = [idx_v[(j * 8 + u) >> 3, pl.ds(((j * 8 + u) & 7) * 16, 16)]
                  for u in range(8)]
            cls = [plsc.scan_count(v) for v in vs]
            for v, (cnt, last) in zip(vs, cls):
                plsc.addupdate_scatter(counts_v, [v], cnt, mask=last)
            return 0

        lax.fori_loop(0, _NUM_IDX // 16 // 8, _hist, 0)
        pltpu.sync_copy(counts_v, out_hbm.at[pl.ds(k * _BINS, _BINS)])


_sc_hist = functools.partial(
    pl.kernel,
    mesh=plsc.VectorSubcoreMesh(core_axis_name="c", subcore_axis_name="s"),
    out_type=jax.ShapeDtypeStruct((_NUM_TABLES * _BINS,), jnp.int32),
    compiler_params=pltpu.CompilerParams(
        use_tc_tiling_on_sc=False, needs_layout_passes=False),
    scratch_types=[
        pltpu.VMEM((128, 128), jnp.int32),
        pltpu.VMEM((_BINS,), jnp.int32),
        pltpu.SemaphoreType.DMA,
    ],
)(_sc_hist_body)


def _tc_body(cnt_ref, tt_ref, out_ref):
    tb = tt_ref[0]
    acc = jnp.zeros((_DIM,), jnp.float32)
    for r in range(_NBLK):
        lo = r * _VBLK
        hi = min((r + 1) * _VBLK, _VOCAB)
        cbr = cnt_ref[0, pl.ds(r, 1), pl.ds(0, hi - lo)]
        acc += jnp.sum(tb[:, lo:hi] * cbr.astype(jnp.float32), axis=1)
    out_ref[0, 0, :] = acc


_tc_reduce = pl.pallas_call(
    _tc_body,
    grid=(_NUM_TABLES,),
    in_specs=[
        pl.BlockSpec((1, _NBLK, _VBLK), lambda k: (k, 0, 0)),
        pl.BlockSpec((1, _DIM, _VOCAB), lambda k: (k, 0, 0)),
    ],
    out_specs=pl.BlockSpec((1, 1, _DIM), lambda k: (k, 0, 0)),
    out_shape=jax.ShapeDtypeStruct((_NUM_TABLES, 1, _DIM), jnp.float32),
    compiler_params=pltpu.CompilerParams(
        dimension_semantics=("arbitrary",)),
)


@jax.jit
def _run(lS_i, tables):
    tt = jnp.swapaxes(tables, 1, 2)
    lsi3 = lS_i.reshape(_NUM_TABLES, 128, 128)
    counts = _sc_hist(lsi3)
    cnt3 = counts.reshape(_NUM_TABLES, _NBLK, _VBLK)
    out = _tc_reduce(cnt3, tt)
    return out.reshape(_NUM_TABLES, _DIM)


def kernel(lS_o, lS_i, tables):
    out = _run(lS_i, tables)
    return tuple(out[k:k + 1] for k in range(_NUM_TABLES))


# speedup vs baseline: 1.2404x; 1.1178x over previous
import functools

import jax
import jax.numpy as jnp
from jax import lax
from jax.experimental import pallas as pl
from jax.experimental.pallas import tpu as pltpu
from jax.experimental.pallas import tpu_sc as plsc

_NUM_TABLES = 26
_VOCAB = 100000
_DIM = 16
_NUM_IDX = 16384
_NC = 2
_NS = 16
_BINS = 102400
_VBLK = 12800
_NBLK = _BINS // _VBLK


def _sc_hist_body(lsi_hbm, out_hbm, idx_v, counts_v, sem):
    k = lax.axis_index("s") * _NC + lax.axis_index("c")

    @pl.when(k < _NUM_TABLES)
    def _():
        h = pltpu.async_copy(lsi_hbm.at[k], idx_v, sem)

        zeros = jnp.zeros((16,), jnp.int32)

        def _zero(j, _):
            for u in range(8):
                counts_v[pl.ds((j * 8 + u) * 16, 16)] = zeros
            return 0

        lax.fori_loop(0, _BINS // 16 // 8, _zero, 0)
        h.wait()

        def _hist(j, _):
            vs = [idx_v[(j * 8 + u) >> 3, pl.ds(((j * 8 + u) & 7) * 16, 16)]
                  for u in range(8)]
            cls = [plsc.scan_count(v) for v in vs]
            for v, (cnt, last) in zip(vs, cls):
                plsc.addupdate_scatter(counts_v, [v], cnt, mask=last)
            return 0

        lax.fori_loop(0, _NUM_IDX // 16 // 8, _hist, 0)
        pltpu.sync_copy(counts_v, out_hbm.at[pl.ds(k * _BINS, _BINS)])


_sc_hist = functools.partial(
    pl.kernel,
    mesh=plsc.VectorSubcoreMesh(core_axis_name="c", subcore_axis_name="s"),
    out_type=jax.ShapeDtypeStruct((_NUM_TABLES * _BINS,), jnp.int32),
    compiler_params=pltpu.CompilerParams(
        use_tc_tiling_on_sc=False, needs_layout_passes=False),
    scratch_types=[
        pltpu.VMEM((128, 128), jnp.int32),
        pltpu.VMEM((_BINS,), jnp.int32),
        pltpu.SemaphoreType.DMA,
    ],
)(_sc_hist_body)


def _tc_body(cnt_ref, tt_ref, out_ref):
    tb = tt_ref[0]
    acc = jnp.zeros((_DIM,), jnp.float32)
    for r in range(_NBLK):
        lo = r * _VBLK
        hi = min((r + 1) * _VBLK, _VOCAB)
        cbr = cnt_ref[pl.ds(lo, hi - lo)]
        acc += jnp.sum(tb[:, lo:hi] * cbr.astype(jnp.float32), axis=1)
    out_ref[0, 0, :] = acc


_tc_reduce = pl.pallas_call(
    _tc_body,
    grid=(_NUM_TABLES,),
    in_specs=[
        pl.BlockSpec((_BINS,), lambda k: (k,)),
        pl.BlockSpec((1, _DIM, _VOCAB), lambda k: (k, 0, 0)),
    ],
    out_specs=pl.BlockSpec((1, 1, _DIM), lambda k: (k, 0, 0)),
    out_shape=jax.ShapeDtypeStruct((_NUM_TABLES, 1, _DIM), jnp.float32),
    compiler_params=pltpu.CompilerParams(
        dimension_semantics=("arbitrary",)),
)


@jax.jit
def _run(lS_i, tables):
    tt = jnp.swapaxes(tables, 1, 2)
    lsi3 = lS_i.reshape(_NUM_TABLES, 128, 128)
    counts = _sc_hist(lsi3)
    out = _tc_reduce(counts, tt)
    return out.reshape(_NUM_TABLES, _DIM)


def kernel(lS_o, lS_i, tables):
    out = _run(lS_i, tables)
    return tuple(out[k:k + 1] for k in range(_NUM_TABLES))


# speedup vs baseline: 1.2484x; 1.0064x over previous
import functools

import jax
import jax.numpy as jnp
from jax import lax
from jax.experimental import pallas as pl
from jax.experimental.pallas import tpu as pltpu
from jax.experimental.pallas import tpu_sc as plsc

_NUM_TABLES = 26
_VOCAB = 100000
_DIM = 16
_NUM_IDX = 16384
_NC = 2
_NS = 16
_BINS = 102400
_VBLK = 12800
_NBLK = _BINS // _VBLK


def _sc_hist_body(lsi_hbm, out_hbm, idx_v, counts_v, sem):
    k = lax.axis_index("s") * _NC + lax.axis_index("c")

    @pl.when(k < _NUM_TABLES)
    def _():
        h = pltpu.async_copy(lsi_hbm.at[k, pl.ds(0, _NUM_IDX)], idx_v, sem)

        zeros = jnp.zeros((16,), jnp.int32)

        def _zero(j, _):
            for u in range(8):
                counts_v[pl.ds((j * 8 + u) * 16, 16)] = zeros
            return 0

        lax.fori_loop(0, _BINS // 16 // 8, _zero, 0)
        h.wait()

        def _hist(j, _):
            vs = [idx_v[pl.ds((j * 8 + u) * 16, 16)] for u in range(8)]
            cls = [plsc.scan_count(v) for v in vs]
            for v, (cnt, last) in zip(vs, cls):
                plsc.addupdate_scatter(counts_v, [v], cnt, mask=last)
            return 0

        lax.fori_loop(0, _NUM_IDX // 16 // 8, _hist, 0)
        pltpu.sync_copy(counts_v, out_hbm.at[pl.ds(k * _BINS, _BINS)])


_sc_hist = functools.partial(
    pl.kernel,
    mesh=plsc.VectorSubcoreMesh(core_axis_name="c", subcore_axis_name="s"),
    out_type=jax.ShapeDtypeStruct((_NUM_TABLES * _BINS,), jnp.int32),
    compiler_params=pltpu.CompilerParams(
        use_tc_tiling_on_sc=False, needs_layout_passes=False),
    scratch_types=[
        pltpu.VMEM((_NUM_IDX,), jnp.int32),
        pltpu.VMEM((_BINS,), jnp.int32),
        pltpu.SemaphoreType.DMA,
    ],
)(_sc_hist_body)


def _tc_body(cnt_ref, tt_ref, out_ref):
    tb = tt_ref[0]
    acc = jnp.zeros((_DIM,), jnp.float32)
    for r in range(_NBLK):
        lo = r * _VBLK
        hi = min((r + 1) * _VBLK, _VOCAB)
        cbr = cnt_ref[pl.ds(lo, hi - lo)]
        acc += jnp.sum(tb[:, lo:hi] * cbr.astype(jnp.float32), axis=1)
    out_ref[0, 0, :] = acc


_tc_reduce = pl.pallas_call(
    _tc_body,
    grid=(_NUM_TABLES,),
    in_specs=[
        pl.BlockSpec((_BINS,), lambda k: (k,)),
        pl.BlockSpec((1, _DIM, _VOCAB), lambda k: (k, 0, 0)),
    ],
    out_specs=pl.BlockSpec((1, 1, _DIM), lambda k: (k, 0, 0)),
    out_shape=jax.ShapeDtypeStruct((_NUM_TABLES, 1, _DIM), jnp.float32),
    compiler_params=pltpu.CompilerParams(
        dimension_semantics=("arbitrary",)),
)


@jax.jit
def _run(lS_i, tables):
    tt = jnp.swapaxes(tables, 1, 2)
    counts = _sc_hist(lS_i)
    out = _tc_reduce(counts, tt)
    return out.reshape(_NUM_TABLES, _DIM)


def kernel(lS_o, lS_i, tables):
    out = _run(lS_i, tables)
    return tuple(out[k:k + 1] for k in range(_NUM_TABLES))
